# baseline (speedup 1.0000x reference)
import functools

import jax
import jax.numpy as jnp
from jax import lax
from jax.experimental import pallas as pl
from jax.experimental.pallas import tpu as pltpu

PADDING_ID = -1
TEMPERATURE = 12.0
D = 64

NPAD = 51200
BN = 2048
EPAD = 802816
BE = 4096

_PREC = jax.lax.Precision.DEFAULT


def _node_enc_body(nf_ref, fid_ref, wa_ref, wc_ref, b_ref, lo_ref, hi_ref):
    nf = nf_ref[...]
    fid = fid_ref[...]
    onehot = (fid == lax.broadcasted_iota(jnp.int32, (BN, 10), 1)).astype(jnp.float32)
    h = jnp.dot(nf, wa_ref[...], precision=_PREC, preferred_element_type=jnp.float32)
    h += jnp.dot(onehot, wc_ref[...], precision=_PREC, preferred_element_type=jnp.float32)
    h = jnp.maximum(h + b_ref[...], 0.0)
    lo_ref[...] = h[:, :32]
    hi_ref[...] = h[:, 32:]


def _node_enc(nf, fid2d, wa, wcomb, b):
    nblk = NPAD // BN
    return pl.pallas_call(
        _node_enc_body,
        grid=(nblk,),
        in_specs=[
            pl.BlockSpec((BN, 5), lambda i: (i, 0)),
            pl.BlockSpec((BN, 1), lambda i: (i, 0)),
            pl.BlockSpec((5, 64), lambda i: (0, 0)),
            pl.BlockSpec((10, 64), lambda i: (0, 0)),
            pl.BlockSpec((1, 64), lambda i: (0, 0)),
        ],
        out_specs=[
            pl.BlockSpec((BN, 32), lambda i: (i, 0)),
            pl.BlockSpec((BN, 32), lambda i: (i, 0)),
        ],
        out_shape=[
            jax.ShapeDtypeStruct((NPAD, 32), jnp.float32),
            jax.ShapeDtypeStruct((NPAD, 32), jnp.float32),
        ],
    )(nf, fid2d, wa, wcomb, b)


def _edge_mlp_body(ea_ref, w1_ref, b1_ref, w2_ref, b2_ref, lo_ref, hi_ref):
    ea = ea_ref[...]
    h = jnp.dot(ea, w1_ref[...], precision=_PREC, preferred_element_type=jnp.float32)
    h = jnp.maximum(h + b1_ref[...], 0.0)
    h = jnp.dot(h, w2_ref[...], precision=_PREC, preferred_element_type=jnp.float32)
    h = h + b2_ref[...]
    lo_ref[...] = h[:, :32]
    hi_ref[...] = h[:, 32:]


def _edge_mlp(ea, w1, b1, w2, b2):
    nblk = EPAD // BE
    return pl.pallas_call(
        _edge_mlp_body,
        grid=(nblk,),
        in_specs=[
            pl.BlockSpec((BE, 5), lambda i: (i, 0)),
            pl.BlockSpec((5, 64), lambda i: (0, 0)),
            pl.BlockSpec((1, 64), lambda i: (0, 0)),
            pl.BlockSpec((64, 64), lambda i: (0, 0)),
            pl.BlockSpec((1, 64), lambda i: (0, 0)),
        ],
        out_specs=[
            pl.BlockSpec((BE, 32), lambda i: (i, 0)),
            pl.BlockSpec((BE, 32), lambda i: (i, 0)),
        ],
        out_shape=[
            jax.ShapeDtypeStruct((EPAD, 32), jnp.float32),
            jax.ShapeDtypeStruct((EPAD, 32), jnp.float32),
        ],
    )(ea, w1, b1, w2, b2)


def _gine_mlp_body(lo_ref, hi_ref, alo_ref, ahi_ref, eps_ref, w1a_ref, w1b_ref,
                   b1_ref, w2_ref, b2_ref, *out_refs):
    scale = 1.0 + eps_ref[0, 0]
    xlo = scale * lo_ref[...] + alo_ref[...]
    xhi = scale * hi_ref[...] + ahi_ref[...]
    u = jnp.dot(xlo, w1a_ref[...], precision=_PREC, preferred_element_type=jnp.float32)
    u += jnp.dot(xhi, w1b_ref[...], precision=_PREC, preferred_element_type=jnp.float32)
    u = jnp.maximum(u + b1_ref[...], 0.0)
    v = jnp.dot(u, w2_ref[...], precision=_PREC, preferred_element_type=jnp.float32)
    h = jnp.maximum(v + b2_ref[...], 0.0)
    if len(out_refs) == 2:
        out_refs[0][...] = h[:, :32]
        out_refs[1][...] = h[:, 32:]
    else:
        nrm = jnp.sqrt(jnp.sum(h * h, axis=-1, keepdims=True))
        out_refs[0][...] = h / jnp.maximum(nrm, 1e-12)


def _gine_mlp(lo, hi, alo, ahi, eps, w1a, w1b, b1, w2, b2, last):
    if last:
        out_specs = [pl.BlockSpec((BN, 64), lambda i: (i, 0))]
        out_shape = [jax.ShapeDtypeStruct((50000, 64), jnp.float32)]
        nblk = (50000 + BN - 1) // BN
    else:
        out_specs = [pl.BlockSpec((BN, 32), lambda i: (i, 0)),
                     pl.BlockSpec((BN, 32), lambda i: (i, 0))]
        out_shape = [jax.ShapeDtypeStruct((NPAD, 32), jnp.float32),
                     jax.ShapeDtypeStruct((NPAD, 32), jnp.float32)]
        nblk = NPAD // BN
    res = pl.pallas_call(
        _gine_mlp_body,
        grid=(nblk,),
        in_specs=[
            pl.BlockSpec((BN, 32), lambda i: (i, 0)),
            pl.BlockSpec((BN, 32), lambda i: (i, 0)),
            pl.BlockSpec((BN, 32), lambda i: (i, 0)),
            pl.BlockSpec((BN, 32), lambda i: (i, 0)),
            pl.BlockSpec((1, 1), lambda i: (0, 0), memory_space=pltpu.SMEM),
            pl.BlockSpec((32, 128), lambda i: (0, 0)),
            pl.BlockSpec((32, 128), lambda i: (0, 0)),
            pl.BlockSpec((1, 128), lambda i: (0, 0)),
            pl.BlockSpec((128, 64), lambda i: (0, 0)),
            pl.BlockSpec((1, 64), lambda i: (0, 0)),
        ],
        out_specs=out_specs,
        out_shape=out_shape,
    )(lo, hi, alo, ahi, eps, w1a, w1b, b1, w2, b2)
    return res


BNL = 2048


def _logits_body(z_ref, hr_ref, out_ref):
    z = z_ref[...]
    hr = hr_ref[...]
    s = lax.dot_general(z, hr, (((1,), (1,)), ((), ())),
                        precision=_PREC, preferred_element_type=jnp.float32)
    out_ref[...] = (TEMPERATURE * s).reshape(8, 50, BNL)


def _logits(z2d, hr):
    n = hr.shape[0]
    nblk = (n + BNL - 1) // BNL
    return pl.pallas_call(
        _logits_body,
        grid=(nblk,),
        in_specs=[
            pl.BlockSpec((400, 64), lambda i: (0, 0)),
            pl.BlockSpec((BNL, 64), lambda i: (i, 0)),
        ],
        out_specs=pl.BlockSpec((8, 50, BNL), lambda i: (0, 0, i)),
        out_shape=jax.ShapeDtypeStruct((8, 50, n), jnp.float32),
    )(z2d, hr)


def _gru_cell(x, h, wih, whh, bih, bhh):
    gi = x @ wih + bih
    gh = h @ whh + bhh
    ir, iz, inn = jnp.split(gi, 3, axis=-1)
    hr, hz, hn = jnp.split(gh, 3, axis=-1)
    r = jax.nn.sigmoid(ir + hr)
    z = jax.nn.sigmoid(iz + hz)
    n = jnp.tanh(inn + r * hn)
    return (1.0 - z) * n + z * h


def kernel(pred_seq, lengths, node_num_feat, floor_id, edge_index, edge_attr,
           teacher_forcing, params):
    N = node_num_feat.shape[0]
    E = edge_index.shape[1]
    B, L = pred_seq.shape

    wa = params["node_W"][:5]
    wcomb = params["floor_emb"] @ params["node_W"][5:]
    hlo, hhi = _node_enc(node_num_feat, floor_id.reshape(N, 1).astype(jnp.int32),
                         wa, wcomb, params["node_b"].reshape(1, 64))

    elo, ehi = _edge_mlp(edge_attr, params["ep_W1"], params["ep_b1"].reshape(1, 64),
                         params["ep_W2"], params["ep_b2"].reshape(1, 64))

    src = jnp.concatenate([edge_index[0], jnp.full((EPAD - E,), N, jnp.int32)])
    dst = jnp.concatenate([edge_index[1], jnp.full((EPAD - E,), N, jnp.int32)])

    for li, lp in enumerate(params["gine"]):
        last = li == len(params["gine"]) - 1
        hfull = jnp.concatenate([hlo[:N], hhi[:N]], axis=1)
        eemb = jnp.concatenate([elo[:E], ehi[:E]], axis=1)
        msg = jnp.maximum(hfull[src[:E]] + eemb, 0.0)
        agg = jax.ops.segment_sum(msg, dst[:E], num_segments=N)
        agg = jnp.pad(agg, ((0, NPAD - N), (0, 0)))
        alo, ahi = agg[:, :32], agg[:, 32:]
        res = _gine_mlp(hlo, hhi, alo, ahi, lp["eps"].reshape(1, 1),
                        lp["W1"][:32], lp["W1"][32:], lp["b1"].reshape(1, 128),
                        lp["W2"], lp["b2"].reshape(1, 64), last)
        if last:
            hr = res[0]
        else:
            hlo, hhi = res

    pred_safe = jnp.where(pred_seq == PADDING_ID, 0, pred_seq)
    enc_inp = hr[pred_safe]
    t_range = jnp.arange(L)
    valid = t_range[None, :] < lengths[:, None]
    h0 = jnp.zeros((B, D), jnp.float32)

    def run_gru(x, h0v, vmask, p):
        xs = (jnp.swapaxes(x, 0, 1), jnp.swapaxes(vmask, 0, 1))

        def step(h, tup):
            xt, vt = tup
            hn = _gru_cell(xt, h, p["Wih"], p["Whh"], p["bih"], p["bhh"])
            h2 = jnp.where(vt[:, None], hn, h)
            return h2, jnp.where(vt[:, None], hn, 0.0)

        _, outs = lax.scan(step, h0v, xs)
        return jnp.swapaxes(outs, 0, 1)

    out_f = run_gru(enc_inp, h0, valid, params["enc_f"])
    idx = jnp.clip(lengths[:, None] - 1 - t_range[None, :], 0, L - 1)
    x_rev = jnp.take_along_axis(enc_inp, idx[:, :, None], axis=1)
    out_r = run_gru(x_rev, h0, valid, params["enc_b"])
    out_b = jnp.take_along_axis(out_r, idx[:, :, None], axis=1) * valid[:, :, None]
    enc_out = jnp.concatenate([out_f, out_b], axis=-1)
    enc_out = enc_out @ params["encproj_W"] + params["encproj_b"]
    nrm = jnp.linalg.norm(enc_out, axis=-1, keepdims=True)
    enc_out = enc_out / jnp.clip(nrm, 1e-12, None)
    mask = pred_seq != PADDING_ID
    denom = jnp.clip(mask.sum(1), 1, None)[:, None].astype(jnp.float32)
    enc_mean = (enc_out * mask[:, :, None]).sum(1) / denom
    tf = jnp.where(teacher_forcing == PADDING_ID, 0, teacher_forcing)
    dec_inp = jnp.concatenate([jnp.zeros((B, 1, D), jnp.float32), hr[tf[:, :-1]]], axis=1)
    dec_out = run_gru(dec_inp, enc_mean, jnp.ones((B, L), dtype=bool), params["dec"])
    scores = jnp.einsum('btd,bld->btl', dec_out, enc_out)
    scores = jnp.where(mask[:, None, :], scores, -1e9)
    attn = jax.nn.softmax(scores, axis=-1)
    ctx = jnp.einsum('btl,bld->btd', attn, enc_out)
    zc = jnp.concatenate([dec_out, ctx], axis=-1) @ params["decout_W"] + params["decout_b"]
    nz = jnp.linalg.norm(zc, axis=-1, keepdims=True)
    z = zc / jnp.clip(nz, 1e-12, None)

    logits = _logits(z.reshape(B * L, D), hr)
    return logits, hr


# speedup vs baseline: 2.4563x; 2.4563x over previous
import functools

import jax
import jax.numpy as jnp
from jax import lax
from jax.experimental import pallas as pl
from jax.experimental.pallas import tpu as pltpu
from jax.experimental.pallas import tpu_sc as plsc

PADDING_ID = -1
TEMPERATURE = 12.0
D = 64

NPAD = 51200
BN = 2048
EPAD = 802816
BE = 4096

_PREC = jax.lax.Precision.DEFAULT


def _node_enc_body(nf_ref, fid_ref, femb_ref, w_ref, b_ref, lo_ref, hi_ref):
    nf = nf_ref[...]
    fid = fid_ref[...]
    onehot = (fid == lax.broadcasted_iota(jnp.int32, (BN, 10), 1)).astype(jnp.float32)
    fe = jnp.dot(onehot, femb_ref[...], precision=jax.lax.Precision.HIGHEST,
                 preferred_element_type=jnp.float32)
    x = jnp.concatenate([nf, fe], axis=-1)
    h = jnp.dot(x, w_ref[...], precision=_PREC, preferred_element_type=jnp.float32)
    h = jnp.maximum(h + b_ref[...], 0.0)
    lo_ref[...] = h[:, :32]
    hi_ref[...] = h[:, 32:]


def _node_enc(nf, fid2d, femb, w, b):
    nblk = NPAD // BN
    return pl.pallas_call(
        _node_enc_body,
        grid=(nblk,),
        in_specs=[
            pl.BlockSpec((BN, 5), lambda i: (i, 0)),
            pl.BlockSpec((BN, 1), lambda i: (i, 0)),
            pl.BlockSpec((10, 8), lambda i: (0, 0)),
            pl.BlockSpec((13, 64), lambda i: (0, 0)),
            pl.BlockSpec((1, 64), lambda i: (0, 0)),
        ],
        out_specs=[
            pl.BlockSpec((BN, 32), lambda i: (i, 0)),
            pl.BlockSpec((BN, 32), lambda i: (i, 0)),
        ],
        out_shape=[
            jax.ShapeDtypeStruct((NPAD, 32), jnp.float32),
            jax.ShapeDtypeStruct((NPAD, 32), jnp.float32),
        ],
    )(nf, fid2d, femb, w, b)


def _edge_mlp_body(ea_ref, w1_ref, b1_ref, w2_ref, b2_ref, lo_ref, hi_ref):
    ea = ea_ref[...]
    h = jnp.dot(ea, w1_ref[...], precision=_PREC, preferred_element_type=jnp.float32)
    h = jnp.maximum(h + b1_ref[...], 0.0)
    h = jnp.dot(h, w2_ref[...], precision=_PREC, preferred_element_type=jnp.float32)
    h = h + b2_ref[...]
    lo_ref[...] = h[:, :32]
    hi_ref[...] = h[:, 32:]


def _edge_mlp(ea, w1, b1, w2, b2):
    nblk = EPAD // BE
    return pl.pallas_call(
        _edge_mlp_body,
        grid=(nblk,),
        in_specs=[
            pl.BlockSpec((BE, 5), lambda i: (i, 0)),
            pl.BlockSpec((5, 64), lambda i: (0, 0)),
            pl.BlockSpec((1, 64), lambda i: (0, 0)),
            pl.BlockSpec((64, 64), lambda i: (0, 0)),
            pl.BlockSpec((1, 64), lambda i: (0, 0)),
        ],
        out_specs=[
            pl.BlockSpec((BE, 32), lambda i: (i, 0)),
            pl.BlockSpec((BE, 32), lambda i: (i, 0)),
        ],
        out_shape=[
            jax.ShapeDtypeStruct((EPAD, 32), jnp.float32),
            jax.ShapeDtypeStruct((EPAD, 32), jnp.float32),
        ],
    )(ea, w1, b1, w2, b2)


def _gine_mlp_body(lo_ref, hi_ref, alo_ref, ahi_ref, eps_ref, w1_ref,
                   b1_ref, w2_ref, b2_ref, *out_refs):
    scale = 1.0 + eps_ref[0, 0]
    xlo = scale * lo_ref[...] + alo_ref[...]
    xhi = scale * hi_ref[...] + ahi_ref[...]
    x = jnp.concatenate([xlo, xhi], axis=-1)
    u = jnp.dot(x, w1_ref[...], precision=_PREC, preferred_element_type=jnp.float32)
    u = jnp.maximum(u + b1_ref[...], 0.0)
    v = jnp.dot(u, w2_ref[...], precision=_PREC, preferred_element_type=jnp.float32)
    h = jnp.maximum(v + b2_ref[...], 0.0)
    if len(out_refs) == 2:
        out_refs[0][...] = h[:, :32]
        out_refs[1][...] = h[:, 32:]
    else:
        nrm = jnp.sqrt(jnp.sum(h * h, axis=-1, keepdims=True))
        out_refs[0][...] = h / jnp.maximum(nrm, 1e-12)


def _gine_mlp(lo, hi, alo, ahi, eps, w1, b1, w2, b2, last):
    if last:
        out_specs = [pl.BlockSpec((BN, 64), lambda i: (i, 0))]
        out_shape = [jax.ShapeDtypeStruct((50000, 64), jnp.float32)]
        nblk = (50000 + BN - 1) // BN
    else:
        out_specs = [pl.BlockSpec((BN, 32), lambda i: (i, 0)),
                     pl.BlockSpec((BN, 32), lambda i: (i, 0))]
        out_shape = [jax.ShapeDtypeStruct((NPAD, 32), jnp.float32),
                     jax.ShapeDtypeStruct((NPAD, 32), jnp.float32)]
        nblk = NPAD // BN
    res = pl.pallas_call(
        _gine_mlp_body,
        grid=(nblk,),
        in_specs=[
            pl.BlockSpec((BN, 32), lambda i: (i, 0)),
            pl.BlockSpec((BN, 32), lambda i: (i, 0)),
            pl.BlockSpec((BN, 32), lambda i: (i, 0)),
            pl.BlockSpec((BN, 32), lambda i: (i, 0)),
            pl.BlockSpec((1, 1), lambda i: (0, 0), memory_space=pltpu.SMEM),
            pl.BlockSpec((64, 128), lambda i: (0, 0)),
            pl.BlockSpec((1, 128), lambda i: (0, 0)),
            pl.BlockSpec((128, 64), lambda i: (0, 0)),
            pl.BlockSpec((1, 64), lambda i: (0, 0)),
        ],
        out_specs=out_specs,
        out_shape=out_shape,
    )(lo, hi, alo, ahi, eps, w1, b1, w2, b2)
    return res


_CB = 2
_CHUNK = 128
_EBLK = _CB * _CHUNK
_EPT = EPAD // 16
_NBLK_T = _EPT // _EBLK
_ROWS_T = NPAD // 16


def _agg_tile_loop(href, eref, src2d, dst2d, acc, idx_s, idx_d, rows_v, eemb_v,
                   sem, tile):
    chunk0 = tile * (_EPT // _CHUNK)

    def block(b, _):
        cbase = chunk0 + b * _CB
        ebase = cbase * _CHUNK
        pltpu.sync_copy(src2d.at[pl.ds(cbase, _CB)], idx_s)
        pltpu.sync_copy(dst2d.at[pl.ds(cbase, _CB)], idx_d)
        cps = [pltpu.async_copy(href.at[idx_s.at[j]],
                                rows_v.at[pl.ds(j * _CHUNK, _CHUNK)], sem)
               for j in range(_CB)]
        pltpu.sync_copy(eref.at[pl.ds(ebase, _EBLK)], eemb_v)
        for cp in cps:
            cp.wait()

        def elem(r, _):
            for k in (0, 16):
                rows_v[r, pl.ds(k, 16)] = jnp.maximum(
                    rows_v[r, pl.ds(k, 16)] + eemb_v[r, pl.ds(k, 16)], 0.0)
            return ()

        lax.fori_loop(0, _EBLK, elem, (), unroll=4)
        for j in range(_CB):
            pltpu.sync_copy(rows_v.at[pl.ds(j * _CHUNK, _CHUNK)],
                            acc.at[idx_d.at[j]], add=True)
        return ()

    lax.fori_loop(0, _NBLK_T, block, ())


def _sc_agg(hlo, hhi, elo, ehi, src2d, dst2d):
    mesh = plsc.VectorSubcoreMesh(core_axis_name="c", subcore_axis_name="s")

    @functools.partial(
        pl.kernel, mesh=mesh,
        compiler_params=pltpu.CompilerParams(use_tc_tiling_on_sc=False),
        out_type=[jax.ShapeDtypeStruct((NPAD, 32), jnp.float32),
                  jax.ShapeDtypeStruct((NPAD, 32), jnp.float32)],
        scratch_types=[
            pltpu.VMEM((_CB, _CHUNK), jnp.int32),
            pltpu.VMEM((_CB, _CHUNK), jnp.int32),
            pltpu.VMEM((_EBLK, 32), jnp.float32),
            pltpu.VMEM((_EBLK, 32), jnp.float32),
            pltpu.VMEM_SHARED((NPAD, 32), jnp.float32),
            pltpu.SemaphoreType.DMA,
        ],
    )
    def k(hlo_h, hhi_h, elo_h, ehi_h, src_h, dst_h, outlo, outhi,
          idx_s, idx_d, rows_v, eemb_v, acc, sem):
        c = lax.axis_index("c")
        s = lax.axis_index("s")

        def zrow(r, _):
            for k2 in (0, 16):
                rows_v[r, pl.ds(k2, 16)] = jnp.zeros((16,), jnp.float32)
            return ()

        lax.fori_loop(0, _EBLK, zrow, (), unroll=4)
        rbase = s * _ROWS_T
        nfull = _ROWS_T // _EBLK
        for kk in range(nfull):
            pltpu.sync_copy(rows_v, acc.at[pl.ds(rbase + kk * _EBLK, _EBLK)])
        rem = _ROWS_T - nfull * _EBLK
        if rem:
            pltpu.sync_copy(rows_v.at[pl.ds(0, rem)],
                            acc.at[pl.ds(rbase + nfull * _EBLK, rem)])
        plsc.subcore_barrier()

        @pl.when(c == 0)
        def _():
            _agg_tile_loop(hlo_h, elo_h, src_h, dst_h, acc, idx_s, idx_d,
                           rows_v, eemb_v, sem, s)

        @pl.when(c == 1)
        def _():
            _agg_tile_loop(hhi_h, ehi_h, src_h, dst_h, acc, idx_s, idx_d,
                           rows_v, eemb_v, sem, s)

        plsc.subcore_barrier()

        @pl.when(c == 0)
        def _():
            pltpu.sync_copy(acc.at[pl.ds(rbase, _ROWS_T)],
                            outlo.at[pl.ds(rbase, _ROWS_T)])

        @pl.when(c == 1)
        def _():
            pltpu.sync_copy(acc.at[pl.ds(rbase, _ROWS_T)],
                            outhi.at[pl.ds(rbase, _ROWS_T)])

    return k(hlo, hhi, elo, ehi, src2d, dst2d)


def _sc_gather_rows(hr, idx2d):
    nch, csz = idx2d.shape
    mesh = plsc.VectorSubcoreMesh(core_axis_name="c", subcore_axis_name="s")

    @functools.partial(
        pl.kernel, mesh=mesh,
        compiler_params=pltpu.CompilerParams(use_tc_tiling_on_sc=False),
        out_type=jax.ShapeDtypeStruct((nch * csz, 64), jnp.float32),
        scratch_types=[
            pltpu.VMEM((nch, csz), jnp.int32),
            pltpu.VMEM((csz, 64), jnp.float32),
            pltpu.SemaphoreType.DMA,
        ],
    )
    def k(hr_h, idx_h, out_h, idx_v, rows_v, sem):
        c = lax.axis_index("c")
        s = lax.axis_index("s")
        wid = s * 2 + c

        @pl.when(wid < nch)
        def _():
            pltpu.sync_copy(idx_h, idx_v)
            pltpu.async_copy(hr_h.at[idx_v.at[wid]], rows_v, sem).wait()
            pltpu.sync_copy(rows_v, out_h.at[pl.ds(wid * csz, csz)])

    return k(hr, idx2d)


def _seq_body(enc_in_ref, dec_in_ref, len_ref, pred_ref,
              fwih_ref, fwhh_ref, fbih_ref, fbhh_ref,
              bwih_ref, bwhh_ref, bbih_ref, bbhh_ref,
              dwih_ref, dwhh_ref, dbih_ref, dbhh_ref,
              epw_ref, epb_ref, dow_ref, dob_ref,
              z_ref, cat_ref, dout_ref):
    L, B = 50, 8

    def cell(xt, h, wih_ref, whh_ref, bih_ref, bhh_ref):
        gi = jnp.dot(xt, wih_ref[...], preferred_element_type=jnp.float32) + bih_ref[...]
        gh = jnp.dot(h, whh_ref[...], preferred_element_type=jnp.float32) + bhh_ref[...]
        r = jax.nn.sigmoid(gi[:, :64] + gh[:, :64])
        z = jax.nn.sigmoid(gi[:, 64:128] + gh[:, 64:128])
        n = jnp.tanh(gi[:, 128:] + r * gh[:, 128:])
        return (1.0 - z) * n + z * h

    lens = len_ref[...]

    def fwd(t, h):
        xt = enc_in_ref[t]
        hn = cell(xt, h, fwih_ref, fwhh_ref, fbih_ref, fbhh_ref)
        vt = t < lens
        h2 = jnp.where(vt, hn, h)
        cat_ref[t, :, :64] = jnp.where(vt, hn, 0.0)
        return h2

    h0 = jnp.zeros((B, 64), jnp.float32)
    lax.fori_loop(0, L, fwd, h0)

    def bwd(i, h):
        p = L - 1 - i
        xt = enc_in_ref[p]
        hn = cell(xt, h, bwih_ref, bwhh_ref, bbih_ref, bbhh_ref)
        vt = p < lens
        h2 = jnp.where(vt, hn, h)
        cat_ref[p, :, 64:] = jnp.where(vt, hn, 0.0)
        return h2

    lax.fori_loop(0, L, bwd, h0)

    enc_cat = cat_ref[...].reshape(L * B, 128)
    eo = jnp.dot(enc_cat, epw_ref[...], preferred_element_type=jnp.float32) + epb_ref[...]
    nrm = jnp.sqrt(jnp.sum(eo * eo, axis=-1, keepdims=True))
    eo = eo / jnp.maximum(nrm, 1e-12)
    enc_out = eo.reshape(L, B, 64)

    maskbl = pred_ref[...] != PADDING_ID
    maskf = maskbl.astype(jnp.float32)
    denom = jnp.maximum(jnp.sum(maskf, axis=1, keepdims=True), 1.0)
    masklb = maskf.T.reshape(L, B, 1)
    enc_mean = jnp.sum(enc_out * masklb, axis=0) / denom

    def dec(t, h):
        xt = dec_in_ref[t]
        hn = cell(xt, h, dwih_ref, dwhh_ref, dbih_ref, dbhh_ref)
        dout_ref[t] = hn
        return hn

    lax.fori_loop(0, L, dec, enc_mean)

    for b in range(8):
        dec_b = dout_ref[:, b, :]
        enc_b = enc_out[:, b, :]
        s = lax.dot_general(dec_b, enc_b, (((1,), (1,)), ((), ())),
                            preferred_element_type=jnp.float32)
        s = jnp.where(maskbl[b][None, :], s, -1e9)
        s = s - jnp.max(s, axis=-1, keepdims=True)
        es = jnp.exp(s)
        attn = es / jnp.sum(es, axis=-1, keepdims=True)
        ctx = jnp.dot(attn, enc_b, preferred_element_type=jnp.float32)
        zc = jnp.dot(jnp.concatenate([dec_b, ctx], axis=1), dow_ref[...],
                     preferred_element_type=jnp.float32) + dob_ref[...]
        zn = jnp.sqrt(jnp.sum(zc * zc, axis=-1, keepdims=True))
        z_ref[b] = zc / jnp.maximum(zn, 1e-12)


def _seq_kernel(enc_in, dec_in, lengths2d, pred_seq, p):
    args = [enc_in, dec_in, lengths2d.astype(jnp.int32), pred_seq.astype(jnp.int32),
            p["enc_f"]["Wih"], p["enc_f"]["Whh"],
            p["enc_f"]["bih"].reshape(1, 192), p["enc_f"]["bhh"].reshape(1, 192),
            p["enc_b"]["Wih"], p["enc_b"]["Whh"],
            p["enc_b"]["bih"].reshape(1, 192), p["enc_b"]["bhh"].reshape(1, 192),
            p["dec"]["Wih"], p["dec"]["Whh"],
            p["dec"]["bih"].reshape(1, 192), p["dec"]["bhh"].reshape(1, 192),
            p["encproj_W"], p["encproj_b"].reshape(1, 64),
            p["decout_W"], p["decout_b"].reshape(1, 64)]
    return pl.pallas_call(
        _seq_body,
        out_shape=jax.ShapeDtypeStruct((8, 50, 64), jnp.float32),
        scratch_shapes=[pltpu.VMEM((50, 8, 128), jnp.float32),
                        pltpu.VMEM((50, 8, 64), jnp.float32)],
    )(*args)


BNL = 2048


def _logits_body(z_ref, hr_ref, out_ref):
    z = z_ref[...]
    hr = hr_ref[...]
    s = lax.dot_general(z, hr, (((1,), (1,)), ((), ())),
                        precision=_PREC, preferred_element_type=jnp.float32)
    out_ref[...] = (TEMPERATURE * s).reshape(8, 50, BNL)


def _logits(z2d, hr):
    n = hr.shape[0]
    nblk = (n + BNL - 1) // BNL
    return pl.pallas_call(
        _logits_body,
        grid=(nblk,),
        in_specs=[
            pl.BlockSpec((400, 64), lambda i: (0, 0)),
            pl.BlockSpec((BNL, 64), lambda i: (i, 0)),
        ],
        out_specs=pl.BlockSpec((8, 50, BNL), lambda i: (0, 0, i)),
        out_shape=jax.ShapeDtypeStruct((8, 50, n), jnp.float32),
    )(z2d, hr)


def _gru_cell(x, h, wih, whh, bih, bhh):
    gi = x @ wih + bih
    gh = h @ whh + bhh
    ir, iz, inn = jnp.split(gi, 3, axis=-1)
    hr, hz, hn = jnp.split(gh, 3, axis=-1)
    r = jax.nn.sigmoid(ir + hr)
    z = jax.nn.sigmoid(iz + hz)
    n = jnp.tanh(inn + r * hn)
    return (1.0 - z) * n + z * h


def kernel(pred_seq, lengths, node_num_feat, floor_id, edge_index, edge_attr,
           teacher_forcing, params):
    N = node_num_feat.shape[0]
    E = edge_index.shape[1]
    B, L = pred_seq.shape

    hlo, hhi = _node_enc(node_num_feat, floor_id.reshape(N, 1).astype(jnp.int32),
                         params["floor_emb"], params["node_W"],
                         params["node_b"].reshape(1, 64))

    elo, ehi = _edge_mlp(edge_attr, params["ep_W1"], params["ep_b1"].reshape(1, 64),
                         params["ep_W2"], params["ep_b2"].reshape(1, 64))

    src = jnp.concatenate([edge_index[0], jnp.full((EPAD - E,), N, jnp.int32)])
    dst = jnp.concatenate([edge_index[1], jnp.full((EPAD - E,), N, jnp.int32)])

    src2d = src.reshape(EPAD // _CHUNK, _CHUNK)
    dst2d = dst.reshape(EPAD // _CHUNK, _CHUNK)
    for li, lp in enumerate(params["gine"]):
        last = li == len(params["gine"]) - 1
        alo, ahi = _sc_agg(hlo, hhi, elo, ehi, src2d, dst2d)
        res = _gine_mlp(hlo, hhi, alo, ahi, lp["eps"].reshape(1, 1),
                        lp["W1"], lp["b1"].reshape(1, 128),
                        lp["W2"], lp["b2"].reshape(1, 64), last)
        if last:
            hr = res[0]
        else:
            hlo, hhi = res

    pred_safe = jnp.where(pred_seq == PADDING_ID, 0, pred_seq).astype(jnp.int32)
    tf = jnp.where(teacher_forcing == PADDING_ID, 0, teacher_forcing).astype(jnp.int32)
    idx_all = jnp.concatenate([pred_safe.T.reshape(-1), tf[:, :-1].T.reshape(-1),
                               jnp.zeros((8,), jnp.int32)])
    rows = _sc_gather_rows(hr, idx_all.reshape(8, 100))
    enc_in = rows[:400].reshape(L, B, D)
    dec_in = jnp.concatenate([jnp.zeros((1, B, D), jnp.float32),
                              rows[400:792].reshape(L - 1, B, D)], axis=0)
    z = _seq_kernel(enc_in, dec_in, lengths.reshape(B, 1), pred_seq, params)

    logits = _logits(z.reshape(B * L, D), hr)
    return logits, hr


# speedup vs baseline: 3.0669x; 1.2486x over previous
import functools

import jax
import jax.numpy as jnp
from jax import lax
from jax.experimental import pallas as pl
from jax.experimental.pallas import tpu as pltpu
from jax.experimental.pallas import tpu_sc as plsc

PADDING_ID = -1
TEMPERATURE = 12.0
D = 64

NPAD = 51200
BN = 2048
EPAD = 802816
BE = 4096

_PREC = jax.lax.Precision.DEFAULT


def _node_enc_body(nf_ref, fid_ref, femb_ref, w_ref, b_ref, lo_ref, hi_ref):
    nf = nf_ref[...]
    fid = fid_ref[...]
    onehot = (fid == lax.broadcasted_iota(jnp.int32, (BN, 10), 1)).astype(jnp.float32)
    fe = jnp.dot(onehot, femb_ref[...], precision=jax.lax.Precision.HIGHEST,
                 preferred_element_type=jnp.float32)
    x = jnp.concatenate([nf, fe], axis=-1)
    h = jnp.dot(x, w_ref[...], precision=_PREC, preferred_element_type=jnp.float32)
    h = jnp.maximum(h + b_ref[...], 0.0)
    lo_ref[...] = h[:, :32]
    hi_ref[...] = h[:, 32:]


def _node_enc(nf, fid2d, femb, w, b):
    nblk = NPAD // BN
    return pl.pallas_call(
        _node_enc_body,
        grid=(nblk,),
        in_specs=[
            pl.BlockSpec((BN, 5), lambda i: (i, 0)),
            pl.BlockSpec((BN, 1), lambda i: (i, 0)),
            pl.BlockSpec((10, 8), lambda i: (0, 0)),
            pl.BlockSpec((13, 64), lambda i: (0, 0)),
            pl.BlockSpec((1, 64), lambda i: (0, 0)),
        ],
        out_specs=[
            pl.BlockSpec((BN, 32), lambda i: (i, 0)),
            pl.BlockSpec((BN, 32), lambda i: (i, 0)),
        ],
        out_shape=[
            jax.ShapeDtypeStruct((NPAD, 32), jnp.float32),
            jax.ShapeDtypeStruct((NPAD, 32), jnp.float32),
        ],
    )(nf, fid2d, femb, w, b)


def _edge_mlp_body(ea_ref, w1_ref, b1_ref, w2_ref, b2_ref, lo_ref, hi_ref):
    ea = ea_ref[...]
    h = jnp.dot(ea, w1_ref[...], precision=_PREC, preferred_element_type=jnp.float32)
    h = jnp.maximum(h + b1_ref[...], 0.0)
    h = jnp.dot(h, w2_ref[...], precision=_PREC, preferred_element_type=jnp.float32)
    h = h + b2_ref[...]
    lo_ref[...] = h[:, :32]
    hi_ref[...] = h[:, 32:]


def _edge_mlp(ea, w1, b1, w2, b2):
    nblk = EPAD // BE
    return pl.pallas_call(
        _edge_mlp_body,
        grid=(nblk,),
        in_specs=[
            pl.BlockSpec((BE, 5), lambda i: (i, 0)),
            pl.BlockSpec((5, 64), lambda i: (0, 0)),
            pl.BlockSpec((1, 64), lambda i: (0, 0)),
            pl.BlockSpec((64, 64), lambda i: (0, 0)),
            pl.BlockSpec((1, 64), lambda i: (0, 0)),
        ],
        out_specs=[
            pl.BlockSpec((BE, 32), lambda i: (i, 0)),
            pl.BlockSpec((BE, 32), lambda i: (i, 0)),
        ],
        out_shape=[
            jax.ShapeDtypeStruct((EPAD, 32), jnp.float32),
            jax.ShapeDtypeStruct((EPAD, 32), jnp.float32),
        ],
    )(ea, w1, b1, w2, b2)


def _gine_mlp_body(lo_ref, hi_ref, alo_ref, ahi_ref, eps_ref, w1_ref,
                   b1_ref, w2_ref, b2_ref, *out_refs):
    scale = 1.0 + eps_ref[0, 0]
    xlo = scale * lo_ref[...] + alo_ref[...]
    xhi = scale * hi_ref[...] + ahi_ref[...]
    x = jnp.concatenate([xlo, xhi], axis=-1)
    u = jnp.dot(x, w1_ref[...], precision=_PREC, preferred_element_type=jnp.float32)
    u = jnp.maximum(u + b1_ref[...], 0.0)
    v = jnp.dot(u, w2_ref[...], precision=_PREC, preferred_element_type=jnp.float32)
    h = jnp.maximum(v + b2_ref[...], 0.0)
    if len(out_refs) == 2:
        out_refs[0][...] = h[:, :32]
        out_refs[1][...] = h[:, 32:]
    else:
        nrm = jnp.sqrt(jnp.sum(h * h, axis=-1, keepdims=True))
        out_refs[0][...] = h / jnp.maximum(nrm, 1e-12)


def _gine_mlp(lo, hi, alo, ahi, eps, w1, b1, w2, b2, last):
    if last:
        out_specs = [pl.BlockSpec((BN, 64), lambda i: (i, 0))]
        out_shape = [jax.ShapeDtypeStruct((50000, 64), jnp.float32)]
        nblk = (50000 + BN - 1) // BN
    else:
        out_specs = [pl.BlockSpec((BN, 32), lambda i: (i, 0)),
                     pl.BlockSpec((BN, 32), lambda i: (i, 0))]
        out_shape = [jax.ShapeDtypeStruct((NPAD, 32), jnp.float32),
                     jax.ShapeDtypeStruct((NPAD, 32), jnp.float32)]
        nblk = NPAD // BN
    res = pl.pallas_call(
        _gine_mlp_body,
        grid=(nblk,),
        in_specs=[
            pl.BlockSpec((BN, 32), lambda i: (i, 0)),
            pl.BlockSpec((BN, 32), lambda i: (i, 0)),
            pl.BlockSpec((BN, 32), lambda i: (i, 0)),
            pl.BlockSpec((BN, 32), lambda i: (i, 0)),
            pl.BlockSpec((1, 1), lambda i: (0, 0), memory_space=pltpu.SMEM),
            pl.BlockSpec((64, 128), lambda i: (0, 0)),
            pl.BlockSpec((1, 128), lambda i: (0, 0)),
            pl.BlockSpec((128, 64), lambda i: (0, 0)),
            pl.BlockSpec((1, 64), lambda i: (0, 0)),
        ],
        out_specs=out_specs,
        out_shape=out_shape,
    )(lo, hi, alo, ahi, eps, w1, b1, w2, b2)
    return res


_CHUNK = 128
_G = 28
_EPT = EPAD // 16
_CPT = _EPT // _CHUNK
_NGRP = _CPT // _G
_EBLK = 256
_ROWS_T = NPAD // 16


def _agg_tile_loop(href, eref, src2d, dst2d, acc, idx_s, idx_d, rows, eemb,
                   gsem, esem, ssem, tile):
    chunk0 = tile * _CPT

    def relu_add(b):
        rv, ev = rows.at[b], eemb.at[b]

        def elem(r, _):
            for k in (0, 16):
                rv[r, pl.ds(k, 16)] = jnp.maximum(
                    rv[r, pl.ds(k, 16)] + ev[r, pl.ds(k, 16)], 0.0)
            return ()

        lax.fori_loop(0, _CHUNK, elem, (), unroll=4)

    def group(g, _):
        cbase = chunk0 + g * _G
        pltpu.sync_copy(src2d.at[pl.ds(cbase, _G)], idx_s)
        pltpu.sync_copy(dst2d.at[pl.ds(cbase, _G)], idx_d)
        gcp, ecp, scp = {}, {}, {}

        def fire(j):
            b = j % 2
            gcp[j] = pltpu.async_copy(href.at[idx_s.at[j]], rows.at[b], gsem)
            ecp[j] = pltpu.async_copy(
                eref.at[pl.ds((cbase + j) * _CHUNK, _CHUNK)], eemb.at[b], esem)

        fire(0)
        for j in range(_G):
            b = j % 2
            if j + 1 < _G:
                if j >= 1:
                    scp[j - 1].wait()
                fire(j + 1)
            gcp[j].wait()
            ecp[j].wait()
            relu_add(b)
            scp[j] = pltpu.async_copy(rows.at[b], acc.at[idx_d.at[j]], ssem,
                                      add=True)
        scp[_G - 2].wait()
        scp[_G - 1].wait()
        return ()

    lax.fori_loop(0, _NGRP, group, ())


def _sc_agg(hlo, hhi, elo, ehi, src2d, dst2d):
    mesh = plsc.VectorSubcoreMesh(core_axis_name="c", subcore_axis_name="s")

    @functools.partial(
        pl.kernel, mesh=mesh,
        compiler_params=pltpu.CompilerParams(use_tc_tiling_on_sc=False),
        out_type=[jax.ShapeDtypeStruct((NPAD, 32), jnp.float32),
                  jax.ShapeDtypeStruct((NPAD, 32), jnp.float32)],
        scratch_types=[
            pltpu.VMEM((_G, _CHUNK), jnp.int32),
            pltpu.VMEM((_G, _CHUNK), jnp.int32),
            pltpu.VMEM((2, _CHUNK, 32), jnp.float32),
            pltpu.VMEM((2, _CHUNK, 32), jnp.float32),
            pltpu.VMEM_SHARED((NPAD, 32), jnp.float32),
            pltpu.SemaphoreType.DMA,
            pltpu.SemaphoreType.DMA,
            pltpu.SemaphoreType.DMA,
        ],
    )
    def k(hlo_h, hhi_h, elo_h, ehi_h, src_h, dst_h, outlo, outhi,
          idx_s, idx_d, rows, eemb, acc, gsem, esem, ssem):
        c = lax.axis_index("c")
        s = lax.axis_index("s")

        zv = rows.at[0]

        def zrow(r, _):
            for k2 in (0, 16):
                zv[r, pl.ds(k2, 16)] = jnp.zeros((16,), jnp.float32)
            return ()

        lax.fori_loop(0, _CHUNK, zrow, (), unroll=4)
        rbase = s * _ROWS_T
        zcps = [pltpu.async_copy(zv, acc.at[pl.ds(rbase + kk * _CHUNK, _CHUNK)],
                                 ssem)
                for kk in range(_ROWS_T // _CHUNK)]
        for cp in zcps:
            cp.wait()
        plsc.subcore_barrier()

        @pl.when(c == 0)
        def _():
            _agg_tile_loop(hlo_h, elo_h, src_h, dst_h, acc, idx_s, idx_d,
                           rows, eemb, gsem, esem, ssem, s)

        @pl.when(c == 1)
        def _():
            _agg_tile_loop(hhi_h, ehi_h, src_h, dst_h, acc, idx_s, idx_d,
                           rows, eemb, gsem, esem, ssem, s)

        plsc.subcore_barrier()

        @pl.when(c == 0)
        def _():
            pltpu.sync_copy(acc.at[pl.ds(rbase, _ROWS_T)],
                            outlo.at[pl.ds(rbase, _ROWS_T)])

        @pl.when(c == 1)
        def _():
            pltpu.sync_copy(acc.at[pl.ds(rbase, _ROWS_T)],
                            outhi.at[pl.ds(rbase, _ROWS_T)])

    return k(hlo, hhi, elo, ehi, src2d, dst2d)


def _sc_gather_rows(hr, idx2d):
    nch, csz = idx2d.shape
    mesh = plsc.VectorSubcoreMesh(core_axis_name="c", subcore_axis_name="s")

    @functools.partial(
        pl.kernel, mesh=mesh,
        compiler_params=pltpu.CompilerParams(use_tc_tiling_on_sc=False),
        out_type=jax.ShapeDtypeStruct((nch * csz, 64), jnp.float32),
        scratch_types=[
            pltpu.VMEM((nch, csz), jnp.int32),
            pltpu.VMEM((csz, 64), jnp.float32),
            pltpu.SemaphoreType.DMA,
        ],
    )
    def k(hr_h, idx_h, out_h, idx_v, rows_v, sem):
        c = lax.axis_index("c")
        s = lax.axis_index("s")
        wid = s * 2 + c

        @pl.when(wid < nch)
        def _():
            pltpu.sync_copy(idx_h, idx_v)
            pltpu.async_copy(hr_h.at[idx_v.at[wid]], rows_v, sem).wait()
            pltpu.sync_copy(rows_v, out_h.at[pl.ds(wid * csz, csz)])

    return k(hr, idx2d)


def _seq_body(enc_in_ref, dec_in_ref, len_ref, pred_ref,
              fwih_ref, fwhh_ref, fbih_ref, fbhh_ref,
              bwih_ref, bwhh_ref, bbih_ref, bbhh_ref,
              dwih_ref, dwhh_ref, dbih_ref, dbhh_ref,
              epw_ref, epb_ref, dow_ref, dob_ref,
              z_ref, cat_ref, dout_ref):
    L, B = 50, 8

    def cell(xt, h, wih_ref, whh_ref, bih_ref, bhh_ref):
        gi = jnp.dot(xt, wih_ref[...], preferred_element_type=jnp.float32) + bih_ref[...]
        gh = jnp.dot(h, whh_ref[...], preferred_element_type=jnp.float32) + bhh_ref[...]
        r = jax.nn.sigmoid(gi[:, :64] + gh[:, :64])
        z = jax.nn.sigmoid(gi[:, 64:128] + gh[:, 64:128])
        n = jnp.tanh(gi[:, 128:] + r * gh[:, 128:])
        return (1.0 - z) * n + z * h

    lens = len_ref[...]

    def fwd(t, h):
        xt = enc_in_ref[t]
        hn = cell(xt, h, fwih_ref, fwhh_ref, fbih_ref, fbhh_ref)
        vt = t < lens
        h2 = jnp.where(vt, hn, h)
        cat_ref[t, :, :64] = jnp.where(vt, hn, 0.0)
        return h2

    h0 = jnp.zeros((B, 64), jnp.float32)
    lax.fori_loop(0, L, fwd, h0)

    def bwd(i, h):
        p = L - 1 - i
        xt = enc_in_ref[p]
        hn = cell(xt, h, bwih_ref, bwhh_ref, bbih_ref, bbhh_ref)
        vt = p < lens
        h2 = jnp.where(vt, hn, h)
        cat_ref[p, :, 64:] = jnp.where(vt, hn, 0.0)
        return h2

    lax.fori_loop(0, L, bwd, h0)

    enc_cat = cat_ref[...].reshape(L * B, 128)
    eo = jnp.dot(enc_cat, epw_ref[...], preferred_element_type=jnp.float32) + epb_ref[...]
    nrm = jnp.sqrt(jnp.sum(eo * eo, axis=-1, keepdims=True))
    eo = eo / jnp.maximum(nrm, 1e-12)
    enc_out = eo.reshape(L, B, 64)

    maskbl = pred_ref[...] != PADDING_ID
    maskf = maskbl.astype(jnp.float32)
    denom = jnp.maximum(jnp.sum(maskf, axis=1, keepdims=True), 1.0)
    masklb = maskf.T.reshape(L, B, 1)
    enc_mean = jnp.sum(enc_out * masklb, axis=0) / denom

    def dec(t, h):
        xt = dec_in_ref[t]
        hn = cell(xt, h, dwih_ref, dwhh_ref, dbih_ref, dbhh_ref)
        dout_ref[t] = hn
        return hn

    lax.fori_loop(0, L, dec, enc_mean)

    for b in range(8):
        dec_b = dout_ref[:, b, :]
        enc_b = enc_out[:, b, :]
        s = lax.dot_general(dec_b, enc_b, (((1,), (1,)), ((), ())),
                            preferred_element_type=jnp.float32)
        s = jnp.where(maskbl[b][None, :], s, -1e9)
        s = s - jnp.max(s, axis=-1, keepdims=True)
        es = jnp.exp(s)
        attn = es / jnp.sum(es, axis=-1, keepdims=True)
        ctx = jnp.dot(attn, enc_b, preferred_element_type=jnp.float32)
        zc = jnp.dot(jnp.concatenate([dec_b, ctx], axis=1), dow_ref[...],
                     preferred_element_type=jnp.float32) + dob_ref[...]
        zn = jnp.sqrt(jnp.sum(zc * zc, axis=-1, keepdims=True))
        z_ref[b] = zc / jnp.maximum(zn, 1e-12)


def _seq_kernel(enc_in, dec_in, lengths2d, pred_seq, p):
    args = [enc_in, dec_in, lengths2d.astype(jnp.int32), pred_seq.astype(jnp.int32),
            p["enc_f"]["Wih"], p["enc_f"]["Whh"],
            p["enc_f"]["bih"].reshape(1, 192), p["enc_f"]["bhh"].reshape(1, 192),
            p["enc_b"]["Wih"], p["enc_b"]["Whh"],
            p["enc_b"]["bih"].reshape(1, 192), p["enc_b"]["bhh"].reshape(1, 192),
            p["dec"]["Wih"], p["dec"]["Whh"],
            p["dec"]["bih"].reshape(1, 192), p["dec"]["bhh"].reshape(1, 192),
            p["encproj_W"], p["encproj_b"].reshape(1, 64),
            p["decout_W"], p["decout_b"].reshape(1, 64)]
    return pl.pallas_call(
        _seq_body,
        out_shape=jax.ShapeDtypeStruct((8, 50, 64), jnp.float32),
        scratch_shapes=[pltpu.VMEM((50, 8, 128), jnp.float32),
                        pltpu.VMEM((50, 8, 64), jnp.float32)],
    )(*args)


BNL = 2048


def _logits_body(z_ref, hr_ref, out_ref):
    z = z_ref[...]
    hr = hr_ref[...]
    s = lax.dot_general(z, hr, (((1,), (1,)), ((), ())),
                        precision=_PREC, preferred_element_type=jnp.float32)
    out_ref[...] = (TEMPERATURE * s).reshape(8, 50, BNL)


def _logits(z2d, hr):
    n = hr.shape[0]
    nblk = (n + BNL - 1) // BNL
    return pl.pallas_call(
        _logits_body,
        grid=(nblk,),
        in_specs=[
            pl.BlockSpec((400, 64), lambda i: (0, 0)),
            pl.BlockSpec((BNL, 64), lambda i: (i, 0)),
        ],
        out_specs=pl.BlockSpec((8, 50, BNL), lambda i: (0, 0, i)),
        out_shape=jax.ShapeDtypeStruct((8, 50, n), jnp.float32),
    )(z2d, hr)


def _gru_cell(x, h, wih, whh, bih, bhh):
    gi = x @ wih + bih
    gh = h @ whh + bhh
    ir, iz, inn = jnp.split(gi, 3, axis=-1)
    hr, hz, hn = jnp.split(gh, 3, axis=-1)
    r = jax.nn.sigmoid(ir + hr)
    z = jax.nn.sigmoid(iz + hz)
    n = jnp.tanh(inn + r * hn)
    return (1.0 - z) * n + z * h


def kernel(pred_seq, lengths, node_num_feat, floor_id, edge_index, edge_attr,
           teacher_forcing, params):
    N = node_num_feat.shape[0]
    E = edge_index.shape[1]
    B, L = pred_seq.shape

    hlo, hhi = _node_enc(node_num_feat, floor_id.reshape(N, 1).astype(jnp.int32),
                         params["floor_emb"], params["node_W"],
                         params["node_b"].reshape(1, 64))

    elo, ehi = _edge_mlp(edge_attr, params["ep_W1"], params["ep_b1"].reshape(1, 64),
                         params["ep_W2"], params["ep_b2"].reshape(1, 64))

    src = jnp.concatenate([edge_index[0], jnp.full((EPAD - E,), N, jnp.int32)])
    dst = jnp.concatenate([edge_index[1], jnp.full((EPAD - E,), N, jnp.int32)])

    src2d = src.reshape(EPAD // _CHUNK, _CHUNK)
    dst2d = dst.reshape(EPAD // _CHUNK, _CHUNK)
    for li, lp in enumerate(params["gine"]):
        last = li == len(params["gine"]) - 1
        alo, ahi = _sc_agg(hlo, hhi, elo, ehi, src2d, dst2d)
        res = _gine_mlp(hlo, hhi, alo, ahi, lp["eps"].reshape(1, 1),
                        lp["W1"], lp["b1"].reshape(1, 128),
                        lp["W2"], lp["b2"].reshape(1, 64), last)
        if last:
            hr = res[0]
        else:
            hlo, hhi = res

    pred_safe = jnp.where(pred_seq == PADDING_ID, 0, pred_seq).astype(jnp.int32)
    tf = jnp.where(teacher_forcing == PADDING_ID, 0, teacher_forcing).astype(jnp.int32)
    idx_all = jnp.concatenate([pred_safe.T.reshape(-1), tf[:, :-1].T.reshape(-1),
                               jnp.zeros((8,), jnp.int32)])
    rows = _sc_gather_rows(hr, idx_all.reshape(8, 100))
    enc_in = rows[:400].reshape(L, B, D)
    dec_in = jnp.concatenate([jnp.zeros((1, B, D), jnp.float32),
                              rows[400:792].reshape(L - 1, B, D)], axis=0)
    z = _seq_kernel(enc_in, dec_in, lengths.reshape(B, 1), pred_seq, params)

    logits = _logits(z.reshape(B * L, D), hr)
    return logits, hr


# speedup vs baseline: 3.9267x; 1.2803x over previous
import functools

import jax
import jax.numpy as jnp
from jax import lax
from jax.experimental import pallas as pl
from jax.experimental.pallas import tpu as pltpu
from jax.experimental.pallas import tpu_sc as plsc

PADDING_ID = -1
TEMPERATURE = 12.0
D = 64

NPAD = 51200
BN = 2048
EPAD = 802816
BE = 4096

_PREC = jax.lax.Precision.DEFAULT


def _node_enc_body(nf_ref, fid_ref, femb_ref, w_ref, b_ref, lo_ref, hi_ref):
    nf_t = nf_ref[...]
    fid = fid_ref[...]
    onehot_t = (fid == lax.broadcasted_iota(jnp.int32, (10, BN), 0)).astype(jnp.float32)
    fe_t = lax.dot_general(femb_ref[...], onehot_t, (((0,), (0,)), ((), ())),
                           precision=jax.lax.Precision.HIGHEST,
                           preferred_element_type=jnp.float32)
    x_t = jnp.concatenate([nf_t, fe_t], axis=0)
    h = lax.dot_general(x_t, w_ref[...], (((0,), (0,)), ((), ())),
                        precision=_PREC, preferred_element_type=jnp.float32)
    h = jnp.maximum(h + b_ref[...], 0.0)
    lo_ref[...] = h[:, :32]
    hi_ref[...] = h[:, 32:]


def _node_enc(nf_t, fid_row, femb, w, b):
    nblk = NPAD // BN
    return pl.pallas_call(
        _node_enc_body,
        grid=(nblk,),
        in_specs=[
            pl.BlockSpec((5, BN), lambda i: (0, i)),
            pl.BlockSpec((1, BN), lambda i: (0, i)),
            pl.BlockSpec((10, 8), lambda i: (0, 0)),
            pl.BlockSpec((13, 64), lambda i: (0, 0)),
            pl.BlockSpec((1, 64), lambda i: (0, 0)),
        ],
        out_specs=[
            pl.BlockSpec((BN, 32), lambda i: (i, 0)),
            pl.BlockSpec((BN, 32), lambda i: (i, 0)),
        ],
        out_shape=[
            jax.ShapeDtypeStruct((NPAD, 32), jnp.float32),
            jax.ShapeDtypeStruct((NPAD, 32), jnp.float32),
        ],
    )(nf_t, fid_row, femb, w, b)


def _edge_mlp_body(ea_ref, w1_ref, b1_ref, w2_ref, b2_ref, lo_ref, hi_ref):
    ea_t = ea_ref[...]
    h = lax.dot_general(ea_t, w1_ref[...], (((0,), (0,)), ((), ())),
                        precision=_PREC, preferred_element_type=jnp.float32)
    h = jnp.maximum(h + b1_ref[...], 0.0)
    h = jnp.dot(h, w2_ref[...], precision=_PREC, preferred_element_type=jnp.float32)
    h = h + b2_ref[...]
    h3 = h.reshape(BE // 4, 4, 64)
    lo_ref[...] = jnp.concatenate([h3[:, k, :32] for k in range(4)], axis=1)
    hi_ref[...] = jnp.concatenate([h3[:, k, 32:] for k in range(4)], axis=1)


def _edge_mlp(ea_t, w1, b1, w2, b2):
    nblk = EPAD // BE
    return pl.pallas_call(
        _edge_mlp_body,
        grid=(nblk,),
        in_specs=[
            pl.BlockSpec((5, BE), lambda i: (0, i)),
            pl.BlockSpec((5, 64), lambda i: (0, 0)),
            pl.BlockSpec((1, 64), lambda i: (0, 0)),
            pl.BlockSpec((64, 64), lambda i: (0, 0)),
            pl.BlockSpec((1, 64), lambda i: (0, 0)),
        ],
        out_specs=[
            pl.BlockSpec((BE // 4, 128), lambda i: (i, 0)),
            pl.BlockSpec((BE // 4, 128), lambda i: (i, 0)),
        ],
        out_shape=[
            jax.ShapeDtypeStruct((EPAD // 4, 128), jnp.float32),
            jax.ShapeDtypeStruct((EPAD // 4, 128), jnp.float32),
        ],
    )(ea_t, w1, b1, w2, b2)


def _gine_mlp_body(lo_ref, hi_ref, alo_ref, ahi_ref, eps_ref, w1_ref,
                   b1_ref, w2_ref, b2_ref, *out_refs):
    scale = 1.0 + eps_ref[0, 0]
    xlo = scale * lo_ref[...] + alo_ref[...]
    xhi = scale * hi_ref[...] + ahi_ref[...]
    x = jnp.concatenate([xlo, xhi], axis=-1)
    u = jnp.dot(x, w1_ref[...], precision=_PREC, preferred_element_type=jnp.float32)
    u = jnp.maximum(u + b1_ref[...], 0.0)
    v = jnp.dot(u, w2_ref[...], precision=_PREC, preferred_element_type=jnp.float32)
    h = jnp.maximum(v + b2_ref[...], 0.0)
    if len(out_refs) == 2:
        out_refs[0][...] = h[:, :32]
        out_refs[1][...] = h[:, 32:]
    else:
        nrm = jnp.sqrt(jnp.sum(h * h, axis=-1, keepdims=True))
        out_refs[0][...] = h / jnp.maximum(nrm, 1e-12)


def _gine_mlp(lo, hi, alo, ahi, eps, w1, b1, w2, b2, last):
    if last:
        out_specs = [pl.BlockSpec((BN, 64), lambda i: (i, 0))]
        out_shape = [jax.ShapeDtypeStruct((50000, 64), jnp.float32)]
        nblk = (50000 + BN - 1) // BN
    else:
        out_specs = [pl.BlockSpec((BN, 32), lambda i: (i, 0)),
                     pl.BlockSpec((BN, 32), lambda i: (i, 0))]
        out_shape = [jax.ShapeDtypeStruct((NPAD, 32), jnp.float32),
                     jax.ShapeDtypeStruct((NPAD, 32), jnp.float32)]
        nblk = NPAD // BN
    res = pl.pallas_call(
        _gine_mlp_body,
        grid=(nblk,),
        in_specs=[
            pl.BlockSpec((BN, 32), lambda i: (i, 0)),
            pl.BlockSpec((BN, 32), lambda i: (i, 0)),
            pl.BlockSpec((BN, 32), lambda i: (i, 0)),
            pl.BlockSpec((BN, 32), lambda i: (i, 0)),
            pl.BlockSpec((1, 1), lambda i: (0, 0), memory_space=pltpu.SMEM),
            pl.BlockSpec((64, 128), lambda i: (0, 0)),
            pl.BlockSpec((1, 128), lambda i: (0, 0)),
            pl.BlockSpec((128, 64), lambda i: (0, 0)),
            pl.BlockSpec((1, 64), lambda i: (0, 0)),
        ],
        out_specs=out_specs,
        out_shape=out_shape,
    )(lo, hi, alo, ahi, eps, w1, b1, w2, b2)
    return res


_CHUNK = 128
_G = 14
_NBUF = 2
_EPT = EPAD // 16
_CPT = _EPT // _CHUNK
_NGRP = _CPT // _G
_ROWS_T = NPAD // 16


def _agg_tile_loop(href, eref, src2d, dst2d, acc, idx_s, idx_d, rows, eemb,
                   gsem, esem, ssem, tile):
    chunk0 = tile * _CPT

    def relu_add(b):
        rv, ev = rows.at[b], eemb.at[b]

        def elem(q, _):
            for j in range(4):
                for h in (0, 16):
                    rv[4 * q + j, pl.ds(h, 16)] = jnp.maximum(
                        rv[4 * q + j, pl.ds(h, 16)]
                        + ev[q, pl.ds(32 * j + h, 16)], 0.0)
            return ()

        lax.fori_loop(0, 32, elem, (), unroll=2)

    def group(g, _):
        cbase = chunk0 + g * _G
        pltpu.sync_copy(src2d.at[pl.ds(cbase, _G)], idx_s)
        pltpu.sync_copy(dst2d.at[pl.ds(cbase, _G)], idx_d)
        gcp, ecp, scp = {}, {}, {}

        def fire(j):
            b = j % _NBUF
            gcp[j] = pltpu.async_copy(href.at[idx_s.at[j]], rows.at[b], gsem)
            ecp[j] = pltpu.async_copy(
                eref.at[pl.ds((cbase + j) * 32, 32)], eemb.at[b], esem)

        fire(0)
        for j in range(_G):
            b = j % _NBUF
            if j + 1 < _G:
                if j >= 1:
                    scp[j - 1].wait()
                fire(j + 1)
            gcp[j].wait()
            ecp[j].wait()
            relu_add(b)
            scp[j] = pltpu.async_copy(rows.at[b], acc.at[idx_d.at[j]], ssem,
                                      add=True)
        scp[_G - 2].wait()
        scp[_G - 1].wait()
        return ()

    lax.fori_loop(0, _NGRP, group, ())


def _sc_agg(hlo, hhi, elo, ehi, src2d, dst2d):
    mesh = plsc.VectorSubcoreMesh(core_axis_name="c", subcore_axis_name="s")

    @functools.partial(
        pl.kernel, mesh=mesh,
        compiler_params=pltpu.CompilerParams(use_tc_tiling_on_sc=False),
        out_type=[jax.ShapeDtypeStruct((NPAD, 32), jnp.float32),
                  jax.ShapeDtypeStruct((NPAD, 32), jnp.float32)],
        scratch_types=[
            pltpu.VMEM((_G, _CHUNK), jnp.int32),
            pltpu.VMEM((_G, _CHUNK), jnp.int32),
            pltpu.VMEM((_NBUF, _CHUNK, 32), jnp.float32),
            pltpu.VMEM((_NBUF, 32, _CHUNK), jnp.float32),
            pltpu.VMEM_SHARED((NPAD, 32), jnp.float32),
            pltpu.SemaphoreType.DMA,
            pltpu.SemaphoreType.DMA,
            pltpu.SemaphoreType.DMA,
        ],
    )
    def k(hlo_h, hhi_h, elo_h, ehi_h, src_h, dst_h, outlo, outhi,
          idx_s, idx_d, rows, eemb, acc, gsem, esem, ssem):
        c = lax.axis_index("c")
        s = lax.axis_index("s")

        zv = rows.at[0]

        def zrow(r, _):
            for k2 in (0, 16):
                zv[r, pl.ds(k2, 16)] = jnp.zeros((16,), jnp.float32)
            return ()

        lax.fori_loop(0, _CHUNK, zrow, (), unroll=4)
        rbase = s * _ROWS_T
        zcps = [pltpu.async_copy(zv, acc.at[pl.ds(rbase + kk * _CHUNK, _CHUNK)],
                                 ssem)
                for kk in range(_ROWS_T // _CHUNK)]
        for cp in zcps:
            cp.wait()
        plsc.subcore_barrier()

        @pl.when(c == 0)
        def _():
            _agg_tile_loop(hlo_h, elo_h, src_h, dst_h, acc, idx_s, idx_d,
                           rows, eemb, gsem, esem, ssem, s)

        @pl.when(c == 1)
        def _():
            _agg_tile_loop(hhi_h, ehi_h, src_h, dst_h, acc, idx_s, idx_d,
                           rows, eemb, gsem, esem, ssem, s)

        plsc.subcore_barrier()

        @pl.when(c == 0)
        def _():
            pltpu.sync_copy(acc.at[pl.ds(rbase, _ROWS_T)],
                            outlo.at[pl.ds(rbase, _ROWS_T)])

        @pl.when(c == 1)
        def _():
            pltpu.sync_copy(acc.at[pl.ds(rbase, _ROWS_T)],
                            outhi.at[pl.ds(rbase, _ROWS_T)])

    return k(hlo, hhi, elo, ehi, src2d, dst2d)


def _sc_gather_rows(hr, idx2d):
    nch, csz = idx2d.shape
    mesh = plsc.VectorSubcoreMesh(core_axis_name="c", subcore_axis_name="s")

    @functools.partial(
        pl.kernel, mesh=mesh,
        compiler_params=pltpu.CompilerParams(use_tc_tiling_on_sc=False),
        out_type=jax.ShapeDtypeStruct((nch * csz, 64), jnp.float32),
        scratch_types=[
            pltpu.VMEM((nch, csz), jnp.int32),
            pltpu.VMEM((csz, 64), jnp.float32),
            pltpu.SemaphoreType.DMA,
        ],
    )
    def k(hr_h, idx_h, out_h, idx_v, rows_v, sem):
        c = lax.axis_index("c")
        s = lax.axis_index("s")
        wid = s * 2 + c

        @pl.when(wid < nch)
        def _():
            pltpu.sync_copy(idx_h, idx_v)
            pltpu.async_copy(hr_h.at[idx_v.at[wid]], rows_v, sem).wait()
            pltpu.sync_copy(rows_v, out_h.at[pl.ds(wid * csz, csz)])

    return k(hr, idx2d)


def _seq_body(enc_in_ref, dec_in_ref, len_ref, pred_ref,
              fwih_ref, fwhh_ref, fbih_ref, fbhh_ref,
              bwih_ref, bwhh_ref, bbih_ref, bbhh_ref,
              dwih_ref, dwhh_ref, dbih_ref, dbhh_ref,
              epw_ref, epb_ref, dow_ref, dob_ref,
              z_ref, cat_ref, dout_ref):
    L, B = 50, 8

    def cell(xt, h, wih_ref, whh_ref, bih_ref, bhh_ref):
        gi = jnp.dot(xt, wih_ref[...], preferred_element_type=jnp.float32) + bih_ref[...]
        gh = jnp.dot(h, whh_ref[...], preferred_element_type=jnp.float32) + bhh_ref[...]
        r = jax.nn.sigmoid(gi[:, :64] + gh[:, :64])
        z = jax.nn.sigmoid(gi[:, 64:128] + gh[:, 64:128])
        n = jnp.tanh(gi[:, 128:] + r * gh[:, 128:])
        return (1.0 - z) * n + z * h

    lens = len_ref[...]

    def fwd(t, h):
        xt = enc_in_ref[t]
        hn = cell(xt, h, fwih_ref, fwhh_ref, fbih_ref, fbhh_ref)
        vt = t < lens
        h2 = jnp.where(vt, hn, h)
        cat_ref[t, :, :64] = jnp.where(vt, hn, 0.0)
        return h2

    h0 = jnp.zeros((B, 64), jnp.float32)
    lax.fori_loop(0, L, fwd, h0)

    def bwd(i, h):
        p = L - 1 - i
        xt = enc_in_ref[p]
        hn = cell(xt, h, bwih_ref, bwhh_ref, bbih_ref, bbhh_ref)
        vt = p < lens
        h2 = jnp.where(vt, hn, h)
        cat_ref[p, :, 64:] = jnp.where(vt, hn, 0.0)
        return h2

    lax.fori_loop(0, L, bwd, h0)

    enc_cat = cat_ref[...].reshape(L * B, 128)
    eo = jnp.dot(enc_cat, epw_ref[...], preferred_element_type=jnp.float32) + epb_ref[...]
    nrm = jnp.sqrt(jnp.sum(eo * eo, axis=-1, keepdims=True))
    eo = eo / jnp.maximum(nrm, 1e-12)
    enc_out = eo.reshape(L, B, 64)

    maskbl = pred_ref[...] != PADDING_ID
    maskf = maskbl.astype(jnp.float32)
    denom = jnp.maximum(jnp.sum(maskf, axis=1, keepdims=True), 1.0)
    masklb = maskf.T.reshape(L, B, 1)
    enc_mean = jnp.sum(enc_out * masklb, axis=0) / denom

    def dec(t, h):
        xt = dec_in_ref[t]
        hn = cell(xt, h, dwih_ref, dwhh_ref, dbih_ref, dbhh_ref)
        dout_ref[t] = hn
        return hn

    lax.fori_loop(0, L, dec, enc_mean)

    for b in range(8):
        dec_b = dout_ref[:, b, :]
        enc_b = enc_out[:, b, :]
        s = lax.dot_general(dec_b, enc_b, (((1,), (1,)), ((), ())),
                            preferred_element_type=jnp.float32)
        s = jnp.where(maskbl[b][None, :], s, -1e9)
        s = s - jnp.max(s, axis=-1, keepdims=True)
        es = jnp.exp(s)
        attn = es / jnp.sum(es, axis=-1, keepdims=True)
        ctx = jnp.dot(attn, enc_b, preferred_element_type=jnp.float32)
        zc = jnp.dot(jnp.concatenate([dec_b, ctx], axis=1), dow_ref[...],
                     preferred_element_type=jnp.float32) + dob_ref[...]
        zn = jnp.sqrt(jnp.sum(zc * zc, axis=-1, keepdims=True))
        z_ref[b] = zc / jnp.maximum(zn, 1e-12)


def _seq_kernel(enc_in, dec_in, lengths2d, pred_seq, p):
    args = [enc_in, dec_in, lengths2d.astype(jnp.int32), pred_seq.astype(jnp.int32),
            p["enc_f"]["Wih"], p["enc_f"]["Whh"],
            p["enc_f"]["bih"].reshape(1, 192), p["enc_f"]["bhh"].reshape(1, 192),
            p["enc_b"]["Wih"], p["enc_b"]["Whh"],
            p["enc_b"]["bih"].reshape(1, 192), p["enc_b"]["bhh"].reshape(1, 192),
            p["dec"]["Wih"], p["dec"]["Whh"],
            p["dec"]["bih"].reshape(1, 192), p["dec"]["bhh"].reshape(1, 192),
            p["encproj_W"], p["encproj_b"].reshape(1, 64),
            p["decout_W"], p["decout_b"].reshape(1, 64)]
    return pl.pallas_call(
        _seq_body,
        out_shape=jax.ShapeDtypeStruct((8, 50, 64), jnp.float32),
        scratch_shapes=[pltpu.VMEM((50, 8, 128), jnp.float32),
                        pltpu.VMEM((50, 8, 64), jnp.float32)],
    )(*args)


BNL = 2048


def _logits_body(z_ref, hr_ref, out_ref):
    z = z_ref[...]
    hr = hr_ref[...]
    s = lax.dot_general(z, hr, (((1,), (1,)), ((), ())),
                        precision=_PREC, preferred_element_type=jnp.float32)
    out_ref[...] = (TEMPERATURE * s).reshape(8, 50, BNL)


def _logits(z2d, hr):
    n = hr.shape[0]
    nblk = (n + BNL - 1) // BNL
    return pl.pallas_call(
        _logits_body,
        grid=(nblk,),
        in_specs=[
            pl.BlockSpec((400, 64), lambda i: (0, 0)),
            pl.BlockSpec((BNL, 64), lambda i: (i, 0)),
        ],
        out_specs=pl.BlockSpec((8, 50, BNL), lambda i: (0, 0, i)),
        out_shape=jax.ShapeDtypeStruct((8, 50, n), jnp.float32),
    )(z2d, hr)


def _gru_cell(x, h, wih, whh, bih, bhh):
    gi = x @ wih + bih
    gh = h @ whh + bhh
    ir, iz, inn = jnp.split(gi, 3, axis=-1)
    hr, hz, hn = jnp.split(gh, 3, axis=-1)
    r = jax.nn.sigmoid(ir + hr)
    z = jax.nn.sigmoid(iz + hz)
    n = jnp.tanh(inn + r * hn)
    return (1.0 - z) * n + z * h


def kernel(pred_seq, lengths, node_num_feat, floor_id, edge_index, edge_attr,
           teacher_forcing, params):
    N = node_num_feat.shape[0]
    E = edge_index.shape[1]
    B, L = pred_seq.shape

    hlo, hhi = _node_enc(node_num_feat.T, floor_id.reshape(1, N).astype(jnp.int32),
                         params["floor_emb"], params["node_W"],
                         params["node_b"].reshape(1, 64))

    elo, ehi = _edge_mlp(edge_attr.T, params["ep_W1"], params["ep_b1"].reshape(1, 64),
                         params["ep_W2"], params["ep_b2"].reshape(1, 64))

    src = jnp.concatenate([edge_index[0], jnp.full((EPAD - E,), N, jnp.int32)])
    dst = jnp.concatenate([edge_index[1], jnp.full((EPAD - E,), N, jnp.int32)])

    src2d = src.reshape(EPAD // _CHUNK, _CHUNK)
    dst2d = dst.reshape(EPAD // _CHUNK, _CHUNK)
    for li, lp in enumerate(params["gine"]):
        last = li == len(params["gine"]) - 1
        alo, ahi = _sc_agg(hlo, hhi, elo, ehi, src2d, dst2d)
        res = _gine_mlp(hlo, hhi, alo, ahi, lp["eps"].reshape(1, 1),
                        lp["W1"], lp["b1"].reshape(1, 128),
                        lp["W2"], lp["b2"].reshape(1, 64), last)
        if last:
            hr = res[0]
        else:
            hlo, hhi = res

    pred_safe = jnp.where(pred_seq == PADDING_ID, 0, pred_seq).astype(jnp.int32)
    tf = jnp.where(teacher_forcing == PADDING_ID, 0, teacher_forcing).astype(jnp.int32)
    idx_all = jnp.concatenate([pred_safe.T.reshape(-1), tf[:, :-1].T.reshape(-1),
                               jnp.zeros((8,), jnp.int32)])
    rows = _sc_gather_rows(hr, idx_all.reshape(8, 100))
    enc_in = rows[:400].reshape(L, B, D)
    dec_in = jnp.concatenate([jnp.zeros((1, B, D), jnp.float32),
                              rows[400:792].reshape(L - 1, B, D)], axis=0)
    z = _seq_kernel(enc_in, dec_in, lengths.reshape(B, 1), pred_seq, params)

    logits = _logits(z.reshape(B * L, D), hr)
    return logits, hr
